# Initial kernel scaffold; baseline (speedup 1.0000x reference)
import jax
import jax.numpy as jnp
from jax.experimental import pallas as pl


def kernel(node_features, edge_features, mask, m_w0, m_b0, m_w1, m_b1, m_w2, m_b2, ln1_w, ln1_b, d_w0, d_b0, d_w1, d_b1, ln2_w, ln2_b):
    raise NotImplementedError("write your pallas kernel here")


# speedup vs baseline: 7.0504x; 7.0504x over previous
import functools

import jax
import jax.numpy as jnp
from jax.experimental import pallas as pl

N, K, D, L = 2048, 48, 128, 3
TN = 256


def _gelu(x):
    return 0.5 * x * (1.0 + jax.lax.erf(x * 0.7071067811865476))


def _ln(x, w, b, eps=1e-5):
    mu = jnp.mean(x, axis=-1, keepdims=True)
    xc = x - mu
    var = jnp.mean(xc * xc, axis=-1, keepdims=True)
    return xc * jax.lax.rsqrt(var + eps) * w + b


def _body(nf_ref, e_ref, mask_ref,
          at_ref, bt_ref, ct_ref, b0_ref, w1t_ref, b1_ref, w2t_ref, b2_ref,
          ln1w_ref, ln1b_ref, dw0t_ref, db0_ref, dw1t_ref, db1_ref,
          ln2w_ref, ln2b_ref, out_ref):
    f32 = jnp.float32
    nf = nf_ref[...]
    e = e_ref[...].reshape(TN * K, D)
    h = nf
    for l in range(L):
        t = (jnp.dot(h, at_ref[l], preferred_element_type=f32)
             + jnp.dot(nf, bt_ref[l], preferred_element_type=f32)
             + b0_ref[l][None, :])
        m = jnp.dot(e, ct_ref[l], preferred_element_type=f32)
        m = _gelu(m.reshape(TN, K, D) + t[:, None, :]).reshape(TN * K, D)
        m = _gelu(jnp.dot(m, w1t_ref[l], preferred_element_type=f32)
                  + b1_ref[l][None, :])
        m = (jnp.dot(m, w2t_ref[l], preferred_element_type=f32)
             + b2_ref[l][None, :])
        agg = jnp.sum(m.reshape(TN, K, D), axis=1) * (1.0 / 30.0)
        h = _ln(h + agg, ln1w_ref[l][None, :], ln1b_ref[l][None, :])
        d = _gelu(jnp.dot(h, dw0t_ref[l], preferred_element_type=f32)
                  + db0_ref[l][None, :])
        h = h + jnp.dot(d, dw1t_ref[l], preferred_element_type=f32) \
            + db1_ref[l][None, :]
        h = _ln(h, ln2w_ref[l][None, :], ln2b_ref[l][None, :])
        h = h * mask_ref[...]
    out_ref[...] = h


@jax.jit
def kernel(node_features, edge_features, mask, m_w0, m_b0, m_w1, m_b1,
           m_w2, m_b2, ln1_w, ln1_b, d_w0, d_b0, d_w1, d_b1, ln2_w, ln2_b):
    at = jnp.transpose(m_w0[:, :, 0:D], (0, 2, 1))
    bt = jnp.transpose(m_w0[:, :, D:2 * D], (0, 2, 1))
    ct = jnp.transpose(m_w0[:, :, 3 * D:4 * D], (0, 2, 1))
    w1t = jnp.transpose(m_w1, (0, 2, 1))
    w2t = jnp.transpose(m_w2, (0, 2, 1))
    dw0t = jnp.transpose(d_w0, (0, 2, 1))
    dw1t = jnp.transpose(d_w1, (0, 2, 1))
    mask2 = mask[:, None]

    grid = (N // TN,)
    node_spec = pl.BlockSpec((TN, D), lambda i: (i, 0))
    edge_spec = pl.BlockSpec((TN, K, D), lambda i: (i, 0, 0))
    mask_spec = pl.BlockSpec((TN, 1), lambda i: (i, 0))

    def full(x):
        nd = x.ndim
        return pl.BlockSpec(x.shape, lambda i, _n=nd: (0,) * _n)

    weights = (at, bt, ct, m_b0, w1t, m_b1, w2t, m_b2,
               ln1_w, ln1_b, dw0t, d_b0, dw1t, d_b1, ln2_w, ln2_b)
    return pl.pallas_call(
        _body,
        grid=grid,
        in_specs=[node_spec, edge_spec, mask_spec] + [full(w) for w in weights],
        out_specs=node_spec,
        out_shape=jax.ShapeDtypeStruct((N, D), jnp.float32),
    )(node_features, edge_features, mask2, *weights)


# speedup vs baseline: 9.3729x; 1.3294x over previous
import jax
import jax.numpy as jnp
from jax.experimental import pallas as pl

N, K, D, L = 2048, 48, 128, 3
TN = 256
R2 = 0.7071067811865476


def _softerf(y):
    return y + y * jax.lax.erf(y)


def _ln(x, w, b, eps=1e-5):
    mu = jnp.mean(x, axis=-1, keepdims=True)
    xc = x - mu
    var = jnp.mean(xc * xc, axis=-1, keepdims=True)
    return xc * jax.lax.rsqrt(var + eps) * w + b


def _body(nf_ref, e_ref, mask_ref,
          at_ref, bt_ref, ct_ref, b0_ref, w1t_ref, b1_ref, w2t_ref, b2_ref,
          ln1w_ref, ln1b_ref, dw0t_ref, db0_ref, dw1t_ref, db1_ref,
          ln2w_ref, ln2b_ref, out_ref):
    f32 = jnp.float32
    nf = nf_ref[...]
    e = e_ref[...].reshape(TN * K, D)
    h = nf
    for l in range(L):
        t = (jnp.dot(h, at_ref[l], preferred_element_type=f32)
             + jnp.dot(nf, bt_ref[l], preferred_element_type=f32)
             + b0_ref[l][None, :])
        y = jnp.dot(e, ct_ref[l], preferred_element_type=f32)
        g = _softerf(y.reshape(TN, K, D) + t[:, None, :]).reshape(TN * K, D)
        y1 = jnp.dot(g, w1t_ref[l], preferred_element_type=f32) \
            + b1_ref[l][None, :]
        s = jnp.sum(_softerf(y1).reshape(TN, K, D), axis=1)
        agg = jnp.dot(s, w2t_ref[l], preferred_element_type=f32) \
            + b2_ref[l][None, :]
        h = _ln(h + agg, ln1w_ref[l][None, :], ln1b_ref[l][None, :])
        yd = jnp.dot(h, dw0t_ref[l], preferred_element_type=f32) \
            + db0_ref[l][None, :]
        h = h + jnp.dot(_softerf(yd), dw1t_ref[l],
                        preferred_element_type=f32) + db1_ref[l][None, :]
        h = _ln(h, ln2w_ref[l][None, :], ln2b_ref[l][None, :])
        h = h * mask_ref[...]
    out_ref[...] = h


@jax.jit
def kernel(node_features, edge_features, mask, m_w0, m_b0, m_w1, m_b1,
           m_w2, m_b2, ln1_w, ln1_b, d_w0, d_b0, d_w1, d_b1, ln2_w, ln2_b):
    at = jnp.transpose(m_w0[:, :, 0:D], (0, 2, 1)) * R2
    bt = jnp.transpose(m_w0[:, :, D:2 * D], (0, 2, 1)) * R2
    ct = jnp.transpose(m_w0[:, :, 3 * D:4 * D], (0, 2, 1)) * R2
    b0 = m_b0 * R2
    w1t = jnp.transpose(m_w1, (0, 2, 1)) * 0.5
    b1 = m_b1 * R2
    w2t = jnp.transpose(m_w2, (0, 2, 1)) * (R2 / 30.0)
    b2 = m_b2 * (K / 30.0)
    dw0t = jnp.transpose(d_w0, (0, 2, 1)) * R2
    db0 = d_b0 * R2
    dw1t = jnp.transpose(d_w1, (0, 2, 1)) * R2
    mask2 = mask[:, None]

    grid = (N // TN,)
    node_spec = pl.BlockSpec((TN, D), lambda i: (i, 0))
    edge_spec = pl.BlockSpec((TN, K, D), lambda i: (i, 0, 0))
    mask_spec = pl.BlockSpec((TN, 1), lambda i: (i, 0))

    def full(x):
        nd = x.ndim
        return pl.BlockSpec(x.shape, lambda i, _n=nd: (0,) * _n)

    weights = (at, bt, ct, b0, w1t, b1, w2t, b2,
               ln1_w, ln1_b, dw0t, db0, dw1t, d_b1, ln2_w, ln2_b)
    return pl.pallas_call(
        _body,
        grid=grid,
        in_specs=[node_spec, edge_spec, mask_spec] + [full(w) for w in weights],
        out_specs=node_spec,
        out_shape=jax.ShapeDtypeStruct((N, D), jnp.float32),
    )(node_features, edge_features, mask2, *weights)
